# Initial kernel scaffold; baseline (speedup 1.0000x reference)
import jax
import jax.numpy as jnp
from jax.experimental import pallas as pl


def kernel(pos, x, idx_neighbour, kernel_points, weight, gamma, beta):
    raise NotImplementedError("write your pallas kernel here")


# speedup vs baseline: 1.1698x; 1.1698x over previous
import functools

import jax
import jax.numpy as jnp
from jax import lax
from jax.experimental import pallas as pl
from jax.experimental.pallas import tpu as pltpu
from jax.experimental.pallas import tpu_sc as plsc

_KP_EXTENT = 1.0
_BN_EPS = 1e-5
_LEAKY_SLOPE = 0.2

_HIGH = lax.Precision.HIGHEST


def _sc_gather(table, idx_flat, row_w):
    nh = idx_flat.shape[0]
    nw = 32
    b_per_w = nh // nw
    chunk = 400
    assert b_per_w % chunk == 0 and b_per_w % 8 == 0

    mesh = plsc.VectorSubcoreMesh(core_axis_name="c", subcore_axis_name="s")

    @functools.partial(
        pl.kernel,
        mesh=mesh,
        out_type=jax.ShapeDtypeStruct((nh, row_w), jnp.float32),
        compiler_params=pltpu.CompilerParams(use_tc_tiling_on_sc=False),
        scratch_types=[
            pltpu.VMEM((chunk,), jnp.int32),
            pltpu.VMEM((chunk, row_w), jnp.float32),
            pltpu.SemaphoreType.DMA,
        ],
    )
    def gather_kernel(table_hbm, idx_hbm, out_hbm, idx_v, rows_v, sem):
        wid = lax.axis_index("s") * 2 + lax.axis_index("c")
        base = wid * b_per_w

        @pl.loop(0, b_per_w, step=chunk)
        def _(off):
            b0 = base + off
            pltpu.sync_copy(idx_hbm.at[pl.ds(b0, chunk)], idx_v)
            pltpu.async_copy(table_hbm.at[idx_v], rows_v, sem).wait()
            pltpu.sync_copy(rows_v, out_hbm.at[pl.ds(b0, chunk)])

    return gather_kernel(table, idx_flat)


def _main_body(nk, h, b, nxp_ref, posp_ref, kpp_ref, wstack_ref,
               out_ref, stats_ref):
    i = pl.program_id(0)
    nxp = nxp_ref[...]
    nx = nxp[:, :128]
    npos = nxp[:, 128:144]
    posb = posp_ref[...]

    pos_rep = jnp.broadcast_to(posb[:, None, :], (b, h, 16)).reshape(b * h, 16)
    rel = npos - pos_rep

    kpp = kpp_ref[...]
    rel_sq = jnp.sum(rel * rel, axis=1, keepdims=True)
    kp_sq = jnp.sum(kpp * kpp, axis=0, keepdims=True)
    dots = lax.dot(rel, kpp, precision=_HIGH)
    sq = jnp.maximum(rel_sq - 2.0 * dots + kp_sq, 0.0)
    wts = jnp.maximum(0.0, 1.0 - jnp.sqrt(sq) / _KP_EXTENT)
    klane = lax.broadcasted_iota(jnp.int32, (1, 128), 1)
    wts = jnp.where(klane < nk, wts, 0.0)

    acc = jnp.zeros((b, 128), jnp.float32)
    for k in range(nk):
        scaled = nx * wts[:, k:k + 1]
        sk = jnp.sum(scaled.reshape(b, h, 128), axis=1)
        wk = wstack_ref[k * 128:(k + 1) * 128, :]
        acc = acc + lax.dot(sk, wk, precision=_HIGH)

    out_ref[...] = acc

    @pl.when(i == 0)
    def _():
        stats_ref[...] = jnp.zeros_like(stats_ref)

    stats_ref[0:1, :] = stats_ref[0:1, :] + jnp.sum(acc, axis=0, keepdims=True)
    stats_ref[1:2, :] = stats_ref[1:2, :] + jnp.sum(acc * acc, axis=0,
                                                    keepdims=True)


def _bn_body(n, raw_ref, stats_ref, gamma_ref, beta_ref, out_ref):
    raw = raw_ref[...]
    mean = stats_ref[0:1, :] * (1.0 / n)
    var = stats_ref[1:2, :] * (1.0 / n) - mean * mean
    a = gamma_ref[...] * lax.rsqrt(var + _BN_EPS)
    shift = beta_ref[...] - mean * a
    y = raw * a + shift
    out_ref[...] = jnp.where(y >= 0.0, y, _LEAKY_SLOPE * y)


def kernel(pos, x, idx_neighbour, kernel_points, weight, gamma, beta):
    n, h = idx_neighbour.shape
    nk = kernel_points.shape[0]
    nh = n * h

    posp = jnp.pad(pos, ((0, 0), (0, 16 - pos.shape[1])))
    table = jnp.concatenate([x, posp], axis=1)
    idx_flat = idx_neighbour.reshape(nh)
    kpp = jnp.zeros((16, 128), jnp.float32).at[:3, :nk].set(kernel_points.T)
    wstack = jnp.zeros((16, 128, 128), jnp.float32).at[:nk].set(
        weight).reshape(16 * 128, 128)

    nxp = _sc_gather(table, idx_flat, 144)

    b = 200
    grid = n // b
    out_raw, stats = pl.pallas_call(
        functools.partial(_main_body, nk, h, b),
        grid=(grid,),
        in_specs=[
            pl.BlockSpec((b * h, 144), lambda i: (i, 0)),
            pl.BlockSpec((b, 16), lambda i: (i, 0)),
            pl.BlockSpec((16, 128), lambda i: (0, 0)),
            pl.BlockSpec((16 * 128, 128), lambda i: (0, 0)),
        ],
        out_specs=[
            pl.BlockSpec((b, 128), lambda i: (i, 0)),
            pl.BlockSpec((8, 128), lambda i: (0, 0)),
        ],
        out_shape=[
            jax.ShapeDtypeStruct((n, 128), jnp.float32),
            jax.ShapeDtypeStruct((8, 128), jnp.float32),
        ],
    )(nxp, posp, kpp, wstack)

    out = pl.pallas_call(
        functools.partial(_bn_body, float(n)),
        in_specs=[
            pl.BlockSpec((n, 128), lambda: (0, 0)),
            pl.BlockSpec((8, 128), lambda: (0, 0)),
            pl.BlockSpec((1, 128), lambda: (0, 0)),
            pl.BlockSpec((1, 128), lambda: (0, 0)),
        ],
        out_specs=pl.BlockSpec((n, 128), lambda: (0, 0)),
        out_shape=jax.ShapeDtypeStruct((n, 128), jnp.float32),
    )(out_raw, stats, gamma.reshape(1, 128), beta.reshape(1, 128))
    return out


# speedup vs baseline: 1.2329x; 1.0540x over previous
import functools

import jax
import jax.numpy as jnp
from jax import lax
from jax.experimental import pallas as pl
from jax.experimental.pallas import tpu as pltpu
from jax.experimental.pallas import tpu_sc as plsc

_KP_EXTENT = 1.0
_BN_EPS = 1e-5
_LEAKY_SLOPE = 0.2

_HIGH = lax.Precision.HIGHEST


def _sc_gather(table, idx_flat, row_w):
    nh = idx_flat.shape[0]
    nw = 32
    b_per_w = nh // nw
    chunk = 400
    assert b_per_w % chunk == 0 and b_per_w % 8 == 0

    mesh = plsc.VectorSubcoreMesh(core_axis_name="c", subcore_axis_name="s")

    @functools.partial(
        pl.kernel,
        mesh=mesh,
        out_type=jax.ShapeDtypeStruct((nh, row_w), jnp.float32),
        compiler_params=pltpu.CompilerParams(use_tc_tiling_on_sc=False),
        scratch_types=[
            pltpu.VMEM((chunk,), jnp.int32),
            pltpu.VMEM((chunk, row_w), jnp.float32),
            pltpu.SemaphoreType.DMA,
        ],
    )
    def gather_kernel(table_hbm, idx_hbm, out_hbm, idx_v, rows_v, sem):
        wid = lax.axis_index("s") * 2 + lax.axis_index("c")
        base = wid * b_per_w

        @pl.loop(0, b_per_w, step=chunk)
        def _(off):
            b0 = base + off
            pltpu.sync_copy(idx_hbm.at[pl.ds(b0, chunk)], idx_v)
            pltpu.async_copy(table_hbm.at[idx_v], rows_v, sem).wait()
            pltpu.sync_copy(rows_v, out_hbm.at[pl.ds(b0, chunk)])

    return gather_kernel(table, idx_flat)


def _main_body(nk, h, b, nxp_ref, posp_ref, kpp_ref, wstack_ref,
               out_ref, stats_ref):
    i = pl.program_id(0)
    nxp = nxp_ref[...]
    nx = nxp[:, :128]
    npos = nxp[:, 128:144]
    posb = posp_ref[...]

    pos_rep = jnp.broadcast_to(posb[:, None, :], (b, h, 16)).reshape(b * h, 16)
    rel = npos - pos_rep

    kpp = kpp_ref[...]
    rel_sq = jnp.sum(rel * rel, axis=1, keepdims=True)
    kp_sq = jnp.sum(kpp * kpp, axis=0, keepdims=True)
    dots = rel[:, 0:1] * kpp[0:1, :]
    dots = dots + rel[:, 1:2] * kpp[1:2, :]
    dots = dots + rel[:, 2:3] * kpp[2:3, :]
    sq = jnp.maximum(rel_sq - 2.0 * dots + kp_sq, 0.0)
    wts = jnp.maximum(0.0, 1.0 - jnp.sqrt(sq) / _KP_EXTENT)
    klane = lax.broadcasted_iota(jnp.int32, (1, 128), 1)
    wts = jnp.where(klane < nk, wts, 0.0)

    sks = []
    for k in range(nk):
        scaled = nx * wts[:, k:k + 1]
        sks.append(jnp.sum(scaled.reshape(b, h, 128), axis=1))
    sk_all = jnp.concatenate(sks, axis=1)
    acc = lax.dot(sk_all, wstack_ref[...], precision=_HIGH)

    out_ref[...] = acc

    @pl.when(i == 0)
    def _():
        stats_ref[...] = jnp.zeros_like(stats_ref)

    stats_ref[0:1, :] = stats_ref[0:1, :] + jnp.sum(acc, axis=0, keepdims=True)
    stats_ref[1:2, :] = stats_ref[1:2, :] + jnp.sum(acc * acc, axis=0,
                                                    keepdims=True)


def _bn_body(n, raw_ref, stats_ref, gamma_ref, beta_ref, out_ref):
    raw = raw_ref[...]
    mean = stats_ref[0:1, :] * (1.0 / n)
    var = stats_ref[1:2, :] * (1.0 / n) - mean * mean
    a = gamma_ref[...] * lax.rsqrt(var + _BN_EPS)
    shift = beta_ref[...] - mean * a
    y = raw * a + shift
    out_ref[...] = jnp.where(y >= 0.0, y, _LEAKY_SLOPE * y)


def kernel(pos, x, idx_neighbour, kernel_points, weight, gamma, beta):
    n, h = idx_neighbour.shape
    nk = kernel_points.shape[0]
    nh = n * h

    posp = jnp.pad(pos, ((0, 0), (0, 16 - pos.shape[1])))
    table = jnp.concatenate([x, posp], axis=1)
    idx_flat = idx_neighbour.reshape(nh)
    kpp = jnp.zeros((16, 128), jnp.float32).at[:3, :nk].set(kernel_points.T)
    wstack = weight.reshape(nk * 128, 128)

    nxp = _sc_gather(table, idx_flat, 144)

    b = 200
    grid = n // b
    out_raw, stats = pl.pallas_call(
        functools.partial(_main_body, nk, h, b),
        grid=(grid,),
        in_specs=[
            pl.BlockSpec((b * h, 144), lambda i: (i, 0)),
            pl.BlockSpec((b, 16), lambda i: (i, 0)),
            pl.BlockSpec((16, 128), lambda i: (0, 0)),
            pl.BlockSpec((nk * 128, 128), lambda i: (0, 0)),
        ],
        out_specs=[
            pl.BlockSpec((b, 128), lambda i: (i, 0)),
            pl.BlockSpec((8, 128), lambda i: (0, 0)),
        ],
        out_shape=[
            jax.ShapeDtypeStruct((n, 128), jnp.float32),
            jax.ShapeDtypeStruct((8, 128), jnp.float32),
        ],
    )(nxp, posp, kpp, wstack)

    out = pl.pallas_call(
        functools.partial(_bn_body, float(n)),
        in_specs=[
            pl.BlockSpec((n, 128), lambda: (0, 0)),
            pl.BlockSpec((8, 128), lambda: (0, 0)),
            pl.BlockSpec((1, 128), lambda: (0, 0)),
            pl.BlockSpec((1, 128), lambda: (0, 0)),
        ],
        out_specs=pl.BlockSpec((n, 128), lambda: (0, 0)),
        out_shape=jax.ShapeDtypeStruct((n, 128), jnp.float32),
    )(out_raw, stats, gamma.reshape(1, 128), beta.reshape(1, 128))
    return out


# speedup vs baseline: 3.0121x; 2.4430x over previous
import functools

import jax
import jax.numpy as jnp
from jax import lax
from jax.experimental import pallas as pl
from jax.experimental.pallas import tpu as pltpu
from jax.experimental.pallas import tpu_sc as plsc

_KP_EXTENT = 1.0
_BN_EPS = 1e-5
_LEAKY_SLOPE = 0.2

_HIGH = lax.Precision.HIGHEST


def _sc_gather(table, idx_flat, row_w):
    nh = idx_flat.shape[0]
    nw = 32
    b_per_w = nh // nw
    chunk = 400
    assert b_per_w % chunk == 0 and b_per_w % 8 == 0

    mesh = plsc.VectorSubcoreMesh(core_axis_name="c", subcore_axis_name="s")

    @functools.partial(
        pl.kernel,
        mesh=mesh,
        out_type=jax.ShapeDtypeStruct((nh, row_w), jnp.float32),
        scratch_types=[
            pltpu.VMEM((chunk,), jnp.int32),
            pltpu.VMEM((chunk, row_w), jnp.float32),
            pltpu.SemaphoreType.DMA,
        ],
    )
    def gather_kernel(table_hbm, idx_hbm, out_hbm, idx_v, rows_v, sem):
        wid = lax.axis_index("s") * 2 + lax.axis_index("c")
        base = wid * b_per_w

        @pl.loop(0, b_per_w, step=chunk)
        def _(off):
            b0 = base + off
            pltpu.sync_copy(idx_hbm.at[pl.ds(b0, chunk)], idx_v)
            pltpu.async_copy(table_hbm.at[idx_v], rows_v, sem).wait()
            pltpu.sync_copy(rows_v, out_hbm.at[pl.ds(b0, chunk)])

    return gather_kernel(table, idx_flat)


def _main_body(nk, h, b, nxp_ref, posp_ref, kpp_ref, wstack_ref,
               out_ref, stats_ref, sk_ref):
    i = pl.program_id(0)
    nx = nxp_ref[:, :128]
    npos = nxp_ref[:, 128:144].reshape(b, h, 16)
    posb = posp_ref[...]

    rel = npos - posb[:, None, :]

    kpp = kpp_ref[...]
    rel_sq = jnp.sum(rel * rel, axis=2, keepdims=True)
    kp_sq = jnp.sum(kpp * kpp, axis=0, keepdims=True)[None]
    dots = rel[:, :, 0:1] * kpp[0:1, :][None]
    dots = dots + rel[:, :, 1:2] * kpp[1:2, :][None]
    dots = dots + rel[:, :, 2:3] * kpp[2:3, :][None]
    sq = jnp.maximum(rel_sq - 2.0 * dots + kp_sq, 0.0)
    wts = jnp.maximum(0.0, 1.0 - jnp.sqrt(sq) / _KP_EXTENT)
    klane = lax.broadcasted_iota(jnp.int32, (1, 1, 128), 2)
    wts = jnp.where(klane < nk, wts, 0.0).reshape(b * h, 128)

    wtsT = wts[:, 0:16].T
    nxb = nx.astype(jnp.bfloat16)
    sub8 = lax.broadcasted_iota(jnp.int32, (128, 256), 0) % 8
    lane32 = lax.broadcasted_iota(jnp.int32, (128, 256), 1) // 32
    keep = sub8 == lane32
    ngroups = (b * h) // 256
    for g in range(ngroups):
        wg = wtsT[:, g * 256:(g + 1) * 256]
        wrep = jnp.broadcast_to(wg[:, None, :], (16, 8, 256)).reshape(128, 256)
        wblk = jnp.where(keep, wrep, 0.0).astype(jnp.bfloat16)
        wf = lax.dot(wblk, nxb[g * 256:(g + 1) * 256, :],
                     preferred_element_type=jnp.float32)
        for k in range(16):
            sk_ref[g * 8:(g + 1) * 8, k * 128:(k + 1) * 128] = (
                wf[k * 8:(k + 1) * 8, :])

    acc = lax.dot(sk_ref[...].astype(jnp.bfloat16), wstack_ref[...],
                  preferred_element_type=jnp.float32)

    out_ref[...] = acc

    @pl.when(i == 0)
    def _():
        stats_ref[...] = jnp.zeros_like(stats_ref)

    stats_ref[0:1, :] = stats_ref[0:1, :] + jnp.sum(acc, axis=0, keepdims=True)
    stats_ref[1:2, :] = stats_ref[1:2, :] + jnp.sum(acc * acc, axis=0,
                                                    keepdims=True)


def _bn_body(n, chunks, raw_ref, stats_ref, gamma_ref, beta_ref, out_ref):
    raw = raw_ref[...]
    stats = jnp.sum(stats_ref[...].reshape(chunks, 8, 128), axis=0)
    mean = stats[0:1, :] * (1.0 / n)
    var = stats[1:2, :] * (1.0 / n) - mean * mean
    a = gamma_ref[...] * lax.rsqrt(var + _BN_EPS)
    shift = beta_ref[...] - mean * a
    y = raw * a + shift
    out_ref[...] = jnp.where(y >= 0.0, y, _LEAKY_SLOPE * y)


def kernel(pos, x, idx_neighbour, kernel_points, weight, gamma, beta):
    n, h = idx_neighbour.shape
    nk = kernel_points.shape[0]
    nh = n * h

    posp = jnp.pad(pos, ((0, 0), (0, 16 - pos.shape[1])))
    table = jnp.pad(jnp.concatenate([x, posp], axis=1),
                    ((0, 0), (0, 112)))
    idx_flat = idx_neighbour.reshape(nh)
    kpp = jnp.zeros((16, 128), jnp.float32).at[:3, :nk].set(kernel_points.T)
    wstack = jnp.zeros((16, 128, 128), jnp.float32).at[:nk].set(
        weight).reshape(16 * 128, 128).astype(jnp.bfloat16)

    nxp = _sc_gather(table, idx_flat, 256)

    b = 200
    grid = n // b
    out_raw, stats = pl.pallas_call(
        functools.partial(_main_body, nk, h, b),
        grid=(grid,),
        in_specs=[
            pl.BlockSpec((b * h, 256), lambda i: (i, 0)),
            pl.BlockSpec((b, 16), lambda i: (i, 0)),
            pl.BlockSpec((16, 128), lambda i: (0, 0)),
            pl.BlockSpec((16 * 128, 128), lambda i: (0, 0)),
        ],
        out_specs=[
            pl.BlockSpec((b, 128), lambda i: (i, 0)),
            pl.BlockSpec((8, 128), lambda i: (0, 0)),
        ],
        out_shape=[
            jax.ShapeDtypeStruct((n, 128), jnp.float32),
            jax.ShapeDtypeStruct((8, 128), jnp.float32),
        ],
        scratch_shapes=[pltpu.VMEM((b, 16 * 128), jnp.float32)],
    )(nxp, posp, kpp, wstack)

    out = pl.pallas_call(
        functools.partial(_bn_body, float(n), 1),
        in_specs=[
            pl.BlockSpec((n, 128), lambda: (0, 0)),
            pl.BlockSpec((8, 128), lambda: (0, 0)),
            pl.BlockSpec((1, 128), lambda: (0, 0)),
            pl.BlockSpec((1, 128), lambda: (0, 0)),
        ],
        out_specs=pl.BlockSpec((n, 128), lambda: (0, 0)),
        out_shape=jax.ShapeDtypeStruct((n, 128), jnp.float32),
    )(out_raw, stats, gamma.reshape(1, 128), beta.reshape(1, 128))
    return out


# speedup vs baseline: 3.0990x; 1.0289x over previous
import functools

import jax
import jax.numpy as jnp
from jax import lax
from jax.experimental import pallas as pl
from jax.experimental.pallas import tpu as pltpu
from jax.experimental.pallas import tpu_sc as plsc

_KP_EXTENT = 1.0
_BN_EPS = 1e-5
_LEAKY_SLOPE = 0.2

_HIGH = lax.Precision.HIGHEST


def _sc_gather(table, idx_flat, row_w):
    nh = idx_flat.shape[0]
    nw = 32
    b_per_w = nh // nw
    chunk = 400
    assert b_per_w % chunk == 0 and b_per_w % 8 == 0

    mesh = plsc.VectorSubcoreMesh(core_axis_name="c", subcore_axis_name="s")

    @functools.partial(
        pl.kernel,
        mesh=mesh,
        out_type=jax.ShapeDtypeStruct((nh, row_w), jnp.float32),
        scratch_types=[
            pltpu.VMEM((chunk,), jnp.int32),
            pltpu.VMEM((chunk, row_w), jnp.float32),
            pltpu.SemaphoreType.DMA,
        ],
    )
    def gather_kernel(table_hbm, idx_hbm, out_hbm, idx_v, rows_v, sem):
        wid = lax.axis_index("s") * 2 + lax.axis_index("c")
        base = wid * b_per_w

        @pl.loop(0, b_per_w, step=chunk)
        def _(off):
            b0 = base + off
            pltpu.sync_copy(idx_hbm.at[pl.ds(b0, chunk)], idx_v)
            pltpu.async_copy(table_hbm.at[idx_v], rows_v, sem).wait()
            pltpu.sync_copy(rows_v, out_hbm.at[pl.ds(b0, chunk)])

    return gather_kernel(table, idx_flat)


def _main_body(nk, h, b, nxp_ref, posrepT_ref, kpc_ref, wstack_ref,
               out_ref, stats_ref, sk_ref):
    i = pl.program_id(0)
    w_i = lax.bitcast_convert_type(nxp_ref[:, 0:64], jnp.int32)
    lo = lax.bitcast_convert_type(w_i << 16, jnp.float32)
    hi = lax.bitcast_convert_type(w_i & jnp.int32(-65536), jnp.float32)
    nx = jnp.concatenate([lo, hi], axis=1)

    srow = lax.broadcasted_iota(jnp.int32, (16, 128), 0)
    slane = lax.broadcasted_iota(jnp.int32, (16, 128), 1)
    emat = jnp.where((slane == srow + 64) | (slane == srow + 80)
                     | (slane == srow + 96), 1.0, 0.0)
    nposT = lax.dot_general(emat, nxp_ref[...],
                            dimension_numbers=(((1,), (1,)), ((), ())),
                            preferred_element_type=jnp.float32)
    relT = nposT - posrepT_ref[...]
    sh = (16, relT.shape[1])
    dx = jnp.broadcast_to(relT[0:1, :], sh) - kpc_ref[:, 0:1]
    dy = jnp.broadcast_to(relT[1:2, :], sh) - kpc_ref[:, 1:2]
    dz = jnp.broadcast_to(relT[2:3, :], sh) - kpc_ref[:, 2:3]
    sqT = dx * dx + dy * dy + dz * dz
    wtsT = jnp.maximum(0.0, 1.0 - jnp.sqrt(sqT) / _KP_EXTENT)
    krow = lax.broadcasted_iota(jnp.int32, (16, 1), 0)
    wtsT = jnp.where(krow < nk, wtsT, 0.0)

    nxb = nx.astype(jnp.bfloat16)
    sub8 = lax.broadcasted_iota(jnp.int32, (128, 256), 0) % 8
    lane32 = lax.broadcasted_iota(jnp.int32, (128, 256), 1) // 32
    keep = sub8 == lane32
    ngroups = (b * h) // 256
    for g in range(ngroups):
        wg = wtsT[:, g * 256:(g + 1) * 256]
        wrep = jnp.broadcast_to(wg[:, None, :], (16, 8, 256)).reshape(128, 256)
        wblk = jnp.where(keep, wrep, 0.0).astype(jnp.bfloat16)
        wf = lax.dot(wblk, nxb[g * 256:(g + 1) * 256, :],
                     preferred_element_type=jnp.float32)
        for k in range(16):
            sk_ref[g * 8:(g + 1) * 8, k * 128:(k + 1) * 128] = (
                wf[k * 8:(k + 1) * 8, :])

    acc = lax.dot(sk_ref[...].astype(jnp.bfloat16), wstack_ref[...],
                  preferred_element_type=jnp.float32)

    out_ref[...] = acc

    @pl.when(i == 0)
    def _():
        stats_ref[...] = jnp.zeros_like(stats_ref)

    stats_ref[0:1, :] = stats_ref[0:1, :] + jnp.sum(acc, axis=0, keepdims=True)
    stats_ref[1:2, :] = stats_ref[1:2, :] + jnp.sum(acc * acc, axis=0,
                                                    keepdims=True)


def _bn_body(n, chunks, raw_ref, stats_ref, gamma_ref, beta_ref, out_ref):
    raw = raw_ref[...]
    stats = jnp.sum(stats_ref[...].reshape(chunks, 8, 128), axis=0)
    mean = stats[0:1, :] * (1.0 / n)
    var = stats[1:2, :] * (1.0 / n) - mean * mean
    a = gamma_ref[...] * lax.rsqrt(var + _BN_EPS)
    shift = beta_ref[...] - mean * a
    y = raw * a + shift
    out_ref[...] = jnp.where(y >= 0.0, y, _LEAKY_SLOPE * y)


def kernel(pos, x, idx_neighbour, kernel_points, weight, gamma, beta):
    n, h = idx_neighbour.shape
    nk = kernel_points.shape[0]
    nh = n * h

    posp = jnp.pad(pos, ((0, 0), (0, 16 - pos.shape[1])))
    xu = lax.bitcast_convert_type(x.astype(jnp.bfloat16),
                                  jnp.uint16).astype(jnp.uint32)
    xw = lax.bitcast_convert_type(xu[:, 0::2] | (xu[:, 1::2] << 16),
                                  jnp.float32)
    posp_hi = posp.astype(jnp.bfloat16).astype(jnp.float32)
    posp_lo = (posp - posp_hi).astype(jnp.bfloat16).astype(jnp.float32)
    posp_l2 = (posp - posp_hi - posp_lo).astype(jnp.bfloat16).astype(
        jnp.float32)
    table = jnp.pad(jnp.concatenate([xw, posp_hi, posp_lo, posp_l2], axis=1),
                    ((0, 0), (0, 16)))
    idx_flat = idx_neighbour.reshape(nh)
    posrepT = jnp.repeat(posp.at[:, 3].set(-1.0).T[:, :, None], h,
                         axis=2).reshape(16, nh)
    kpc = jnp.zeros((16, 128), jnp.float32).at[:nk, 0:3].set(kernel_points)
    perm = jnp.concatenate([jnp.arange(0, 128, 2), jnp.arange(1, 128, 2)])
    wstack = jnp.zeros((16, 128, 128), jnp.float32).at[:nk].set(
        weight[:, perm, :]).reshape(16 * 128, 128).astype(jnp.bfloat16)

    nxp = _sc_gather(table, idx_flat, 128)

    b = 200
    grid = n // b
    out_raw, stats = pl.pallas_call(
        functools.partial(_main_body, nk, h, b),
        grid=(grid,),
        in_specs=[
            pl.BlockSpec((b * h, 128), lambda i: (i, 0)),
            pl.BlockSpec((16, b * h), lambda i: (0, i)),
            pl.BlockSpec((16, 128), lambda i: (0, 0)),
            pl.BlockSpec((16 * 128, 128), lambda i: (0, 0)),
        ],
        out_specs=[
            pl.BlockSpec((b, 128), lambda i: (i, 0)),
            pl.BlockSpec((8, 128), lambda i: (0, 0)),
        ],
        out_shape=[
            jax.ShapeDtypeStruct((n, 128), jnp.float32),
            jax.ShapeDtypeStruct((8, 128), jnp.float32),
        ],
        scratch_shapes=[pltpu.VMEM((b, 16 * 128), jnp.float32)],
    )(nxp, posrepT, kpc, wstack)

    out = pl.pallas_call(
        functools.partial(_bn_body, float(n), 1),
        in_specs=[
            pl.BlockSpec((n, 128), lambda: (0, 0)),
            pl.BlockSpec((8, 128), lambda: (0, 0)),
            pl.BlockSpec((1, 128), lambda: (0, 0)),
            pl.BlockSpec((1, 128), lambda: (0, 0)),
        ],
        out_specs=pl.BlockSpec((n, 128), lambda: (0, 0)),
        out_shape=jax.ShapeDtypeStruct((n, 128), jnp.float32),
    )(out_raw, stats, gamma.reshape(1, 128), beta.reshape(1, 128))
    return out
